# baseline (speedup 1.0000x reference)
import functools

import numpy as np

import jax
import jax.numpy as jnp
from jax.experimental import pallas as pl
from jax.experimental.pallas import tpu as pltpu

_BF16 = jnp.bfloat16
_F32 = jnp.float32


@functools.lru_cache(maxsize=None)
def _interp_mat_np(out_size, in_size):
    out_size, in_size = int(out_size), int(in_size)
    if out_size == in_size:
        return np.eye(out_size, dtype=np.float32)
    scale = in_size / out_size
    src = np.maximum((np.arange(out_size, dtype=np.float64) + 0.5) * scale - 0.5, 0.0)
    i0 = np.clip(np.floor(src).astype(np.int64), 0, in_size - 1)
    i1 = np.minimum(i0 + 1, in_size - 1)
    frac = (src - i0).astype(np.float32)
    m = np.zeros((out_size, in_size), dtype=np.float32)
    rows = np.arange(out_size)
    np.add.at(m, (rows, i0), 1.0 - frac)
    np.add.at(m, (rows, i1), frac)
    return m


@functools.lru_cache(maxsize=None)
def _resize_lhs_np(in_hw, out_hw):
    (hin, win), (ho, wo) = in_hw, out_hw
    a = _interp_mat_np(int(ho), int(hin))
    b = _interp_mat_np(int(wo), int(win))
    return np.ascontiguousarray(np.kron(a, b))


def _make_body(offs):
    (o0, o1, o2, o3, o4, o5, o6, o7, o8, o9) = offs

    def _neck_body(c1_ref, x2_ref, x3_ref, x4_ref, x5_ref,
                   r4_ref, r12_ref, r23_ref, r34_ref,
                   w_ref,
                   p1_ref, p2_ref, p3_ref, p4_ref, p5_ref):
        def brow(i):
            return w_ref[o9 + i:o9 + i + 1, :].astype(_F32)
        hw = p5_ref.shape[1]
        y5 = jax.lax.dot_general(
            c1_ref[0].astype(_BF16), w_ref[o0:o1, :],
            (((0,), (0,)), ((), ())), preferred_element_type=_F32)
        y5 = y5.reshape(hw, y5.shape[-1]) + brow(0)
        p5_ref[0] = y5.astype(p5_ref.dtype)

        d = jnp.dot(x2_ref[0].astype(_BF16), w_ref[o1:o2, :],
                    preferred_element_type=_F32) + brow(1)
        y4 = jnp.dot(r4_ref[...], d.astype(_BF16), preferred_element_type=_F32)
        p4_ref[0] = y4.astype(p4_ref.dtype)

        d1 = jnp.dot(x5_ref[0].astype(_BF16), w_ref[o2:o3, :],
                     preferred_element_type=_F32) + brow(2)
        d2 = jnp.dot(x4_ref[0].astype(_BF16), w_ref[o3:o4, :],
                     preferred_element_type=_F32) + brow(3)
        d3 = jnp.dot(x3_ref[0].astype(_BF16), w_ref[o4:o5, :],
                     preferred_element_type=_F32) + brow(4)
        p1 = jnp.dot(r12_ref[...], d1.astype(_BF16), preferred_element_type=_F32)
        d2_2 = jnp.maximum(
            jnp.dot(p1.astype(_BF16), w_ref[o5:o6, :], preferred_element_type=_F32)
            + jnp.dot(d2.astype(_BF16), w_ref[o6:o7, :], preferred_element_type=_F32)
            + brow(5), 0.0)
        p2 = jnp.dot(r23_ref[...], d2_2.astype(_BF16), preferred_element_type=_F32)
        d3_2 = jnp.maximum(
            jnp.dot(p2.astype(_BF16), w_ref[o7:o8, :], preferred_element_type=_F32)
            + jnp.dot(d3.astype(_BF16), w_ref[o8:o9, :], preferred_element_type=_F32)
            + brow(6), 0.0)
        p3 = jnp.dot(r34_ref[...], d3_2.astype(_BF16), preferred_element_type=_F32)
        p1_ref[0] = p1.astype(p1_ref.dtype)
        p2_ref[0] = p2.astype(p2_ref.dtype)
        p3_ref[0] = p3.astype(p3_ref.dtype)

    return _neck_body


def kernel(c1, c2, c3, c4, c5,
           dsn0_w, dsn0_b, dsn1_w, dsn1_b, dsn2_w, dsn2_b, dsn3_w, dsn3_b,
           dsn4_w, dsn4_b, cat0_wa, cat0_wb, cat0_b, cat1_wa, cat1_wb, cat1_b):
    N, ci1, H1, W1 = c1.shape
    _, ci2, H2, W2 = c2.shape
    _, ci3, H3, W3 = c3.shape
    _, ci4, H4, W4 = c4.shape
    _, ci5, H5, W5 = c5.shape
    C = dsn0_w.shape[0]
    dt = c1.dtype
    hw1, hw2, hw3, hw4, hw5 = H1 * W1, H2 * W2, H3 * W3, H4 * W4, H5 * W5

    x2 = c2.transpose(0, 2, 3, 1).reshape(N, hw2, ci2)
    x3 = c3.transpose(0, 2, 3, 1).reshape(N, hw3, ci3)
    x4 = c4.transpose(0, 2, 3, 1).reshape(N, hw4, ci4)
    x5 = c5.transpose(0, 2, 3, 1).reshape(N, hw5, ci5)

    wlist = [dsn0_w, dsn1_w, dsn4_w, dsn3_w, dsn2_w,
             cat0_wa, cat0_wb, cat1_wa, cat1_wb]
    offs = [0]
    for w in wlist:
        offs.append(offs[-1] + w.shape[1])
    blist = [dsn0_b, dsn1_b, dsn4_b, dsn3_b, dsn2_b, cat0_b, cat1_b]
    wcat = jnp.concatenate([w.T for w in wlist] + [b.T for b in blist],
                           axis=0).astype(_BF16)

    r4 = jnp.asarray(_resize_lhs_np((H2, W2), (H1, W1))).astype(_BF16)
    r12 = jnp.asarray(_resize_lhs_np((H5, W5), (H4, W4))).astype(_BF16)
    r23 = jnp.asarray(_resize_lhs_np((H4, W4), (H3, W3))).astype(_BF16)
    r34 = jnp.asarray(_resize_lhs_np((H3, W3), (H2, W2))).astype(_BF16)

    def full(shape):
        return pl.BlockSpec(shape, lambda n: (0,) * len(shape))

    p1f, p2f, p3f, p4f, p5f = pl.pallas_call(
        _make_body(tuple(offs)),
        out_shape=(jax.ShapeDtypeStruct((N, hw4, C), dt),
                   jax.ShapeDtypeStruct((N, hw3, C), dt),
                   jax.ShapeDtypeStruct((N, hw2, C), dt),
                   jax.ShapeDtypeStruct((N, hw1, C), dt),
                   jax.ShapeDtypeStruct((N, hw1, C), dt)),
        grid=(N,),
        in_specs=[
            pl.BlockSpec((1, ci1, H1, W1), lambda n: (n, 0, 0, 0)),
            pl.BlockSpec((1, hw2, ci2), lambda n: (n, 0, 0)),
            pl.BlockSpec((1, hw3, ci3), lambda n: (n, 0, 0)),
            pl.BlockSpec((1, hw4, ci4), lambda n: (n, 0, 0)),
            pl.BlockSpec((1, hw5, ci5), lambda n: (n, 0, 0)),
            full((hw1, hw2)), full((hw4, hw5)), full((hw3, hw4)), full((hw2, hw3)),
            full((offs[-1] + 7, C)),
        ],
        out_specs=(pl.BlockSpec((1, hw4, C), lambda n: (n, 0, 0)),
                   pl.BlockSpec((1, hw3, C), lambda n: (n, 0, 0)),
                   pl.BlockSpec((1, hw2, C), lambda n: (n, 0, 0)),
                   pl.BlockSpec((1, hw1, C), lambda n: (n, 0, 0)),
                   pl.BlockSpec((1, hw1, C), lambda n: (n, 0, 0))),
        compiler_params=pltpu.CompilerParams(
            dimension_semantics=("parallel",),
            vmem_limit_bytes=56 * 1024 * 1024),
    )(c1, x2, x3, x4, x5,
      r4, r12, r23, r34, wcat)

    def to_nchw(p, h, w):
        return p.reshape(N, h, w, C).transpose(0, 3, 1, 2)

    return (to_nchw(p1f, H4, W4),
            to_nchw(p2f, H3, W3),
            to_nchw(p3f, H2, W2),
            to_nchw(p4f, H1, W1),
            to_nchw(p5f, H1, W1))


# speedup vs baseline: 1.1361x; 1.1361x over previous
import functools

import numpy as np

import jax
import jax.numpy as jnp
from jax.experimental import pallas as pl
from jax.experimental.pallas import tpu as pltpu

_BF16 = jnp.bfloat16
_F32 = jnp.float32


@functools.lru_cache(maxsize=None)
def _interp_mat_np(out_size, in_size):
    out_size, in_size = int(out_size), int(in_size)
    if out_size == in_size:
        return np.eye(out_size, dtype=np.float32)
    scale = in_size / out_size
    src = np.maximum((np.arange(out_size, dtype=np.float64) + 0.5) * scale - 0.5, 0.0)
    i0 = np.clip(np.floor(src).astype(np.int64), 0, in_size - 1)
    i1 = np.minimum(i0 + 1, in_size - 1)
    frac = (src - i0).astype(np.float32)
    m = np.zeros((out_size, in_size), dtype=np.float32)
    rows = np.arange(out_size)
    np.add.at(m, (rows, i0), 1.0 - frac)
    np.add.at(m, (rows, i1), frac)
    return m


@functools.lru_cache(maxsize=None)
def _resize_lhs_np(in_hw, out_hw):
    (hin, win), (ho, wo) = in_hw, out_hw
    a = _interp_mat_np(int(ho), int(hin))
    b = _interp_mat_np(int(wo), int(win))
    return np.ascontiguousarray(np.kron(a, b))


def _make_body(offs):
    (o0, o1, o2, o3, o4, o5, o6, o7, o8, o9) = offs

    def _neck_body(c1_ref, x2_ref, x3_ref, x4_ref, x5_ref,
                   r4_ref, r12_ref, r23_ref, r34_ref,
                   w_ref, b_ref,
                   p1_ref, p2_ref, p3_ref, p4_ref, p5_ref):
        d1 = jnp.dot(x5_ref[0].astype(_BF16), w_ref[o2:o3, :],
                     preferred_element_type=_F32) + b_ref[2:3, :]
        d2 = jnp.dot(x4_ref[0].astype(_BF16), w_ref[o3:o4, :],
                     preferred_element_type=_F32) + b_ref[3:4, :]
        d3 = jnp.dot(x3_ref[0].astype(_BF16), w_ref[o4:o5, :],
                     preferred_element_type=_F32) + b_ref[4:5, :]
        p1 = jnp.dot(r12_ref[...], d1.astype(_BF16), preferred_element_type=_F32)
        d2_2 = jnp.maximum(
            jnp.dot(p1.astype(_BF16), w_ref[o5:o6, :], preferred_element_type=_F32)
            + jnp.dot(d2.astype(_BF16), w_ref[o6:o7, :], preferred_element_type=_F32)
            + b_ref[5:6, :], 0.0)
        p2 = jnp.dot(r23_ref[...], d2_2.astype(_BF16), preferred_element_type=_F32)
        d3_2 = jnp.maximum(
            jnp.dot(p2.astype(_BF16), w_ref[o7:o8, :], preferred_element_type=_F32)
            + jnp.dot(d3.astype(_BF16), w_ref[o8:o9, :], preferred_element_type=_F32)
            + b_ref[6:7, :], 0.0)
        p3 = jnp.dot(r34_ref[...], d3_2.astype(_BF16), preferred_element_type=_F32)
        p1_ref[0] = p1.astype(p1_ref.dtype)
        p2_ref[0] = p2.astype(p2_ref.dtype)
        p3_ref[0] = p3.astype(p3_ref.dtype)

        d = jnp.dot(x2_ref[0].astype(_BF16), w_ref[o1:o2, :],
                    preferred_element_type=_F32) + b_ref[1:2, :]
        y4 = jnp.dot(r4_ref[...], d.astype(_BF16), preferred_element_type=_F32)
        p4_ref[0] = y4.astype(p4_ref.dtype)

        hw = p5_ref.shape[1]
        y5 = jax.lax.dot_general(
            c1_ref[0].astype(_BF16), w_ref[o0:o1, :],
            (((0,), (0,)), ((), ())), preferred_element_type=_F32)
        y5 = y5.reshape(hw, y5.shape[-1]) + b_ref[0:1, :]
        p5_ref[0] = y5.astype(p5_ref.dtype)

    return _neck_body


def kernel(c1, c2, c3, c4, c5,
           dsn0_w, dsn0_b, dsn1_w, dsn1_b, dsn2_w, dsn2_b, dsn3_w, dsn3_b,
           dsn4_w, dsn4_b, cat0_wa, cat0_wb, cat0_b, cat1_wa, cat1_wb, cat1_b):
    N, ci1, H1, W1 = c1.shape
    _, ci2, H2, W2 = c2.shape
    _, ci3, H3, W3 = c3.shape
    _, ci4, H4, W4 = c4.shape
    _, ci5, H5, W5 = c5.shape
    C = dsn0_w.shape[0]
    dt = c1.dtype
    hw1, hw2, hw3, hw4, hw5 = H1 * W1, H2 * W2, H3 * W3, H4 * W4, H5 * W5

    x2 = c2.transpose(0, 2, 3, 1).reshape(N, hw2, ci2)
    x3 = c3.transpose(0, 2, 3, 1).reshape(N, hw3, ci3)
    x4 = c4.transpose(0, 2, 3, 1).reshape(N, hw4, ci4)
    x5 = c5.transpose(0, 2, 3, 1).reshape(N, hw5, ci5)

    wlist = [dsn0_w, dsn1_w, dsn4_w, dsn3_w, dsn2_w,
             cat0_wa, cat0_wb, cat1_wa, cat1_wb]
    offs = [0]
    for w in wlist:
        offs.append(offs[-1] + w.shape[1])
    wcat = jnp.concatenate([w.T for w in wlist], axis=0).astype(_BF16)
    bcat = jnp.concatenate([dsn0_b.T, dsn1_b.T, dsn4_b.T, dsn3_b.T, dsn2_b.T,
                            cat0_b.T, cat1_b.T], axis=0)

    r4 = jnp.asarray(_resize_lhs_np((H2, W2), (H1, W1))).astype(_BF16)
    r12 = jnp.asarray(_resize_lhs_np((H5, W5), (H4, W4))).astype(_BF16)
    r23 = jnp.asarray(_resize_lhs_np((H4, W4), (H3, W3))).astype(_BF16)
    r34 = jnp.asarray(_resize_lhs_np((H3, W3), (H2, W2))).astype(_BF16)

    def full(shape):
        return pl.BlockSpec(shape, lambda n: (0,) * len(shape))

    p1f, p2f, p3f, p4f, p5f = pl.pallas_call(
        _make_body(tuple(offs)),
        out_shape=(jax.ShapeDtypeStruct((N, hw4, C), dt),
                   jax.ShapeDtypeStruct((N, hw3, C), dt),
                   jax.ShapeDtypeStruct((N, hw2, C), dt),
                   jax.ShapeDtypeStruct((N, hw1, C), dt),
                   jax.ShapeDtypeStruct((N, hw1, C), dt)),
        grid=(N,),
        in_specs=[
            pl.BlockSpec((1, ci1, H1, W1), lambda n: (n, 0, 0, 0)),
            pl.BlockSpec((1, hw2, ci2), lambda n: (n, 0, 0)),
            pl.BlockSpec((1, hw3, ci3), lambda n: (n, 0, 0)),
            pl.BlockSpec((1, hw4, ci4), lambda n: (n, 0, 0)),
            pl.BlockSpec((1, hw5, ci5), lambda n: (n, 0, 0)),
            full((hw1, hw2)), full((hw4, hw5)), full((hw3, hw4)), full((hw2, hw3)),
            full((offs[-1], C)), full((7, C)),
        ],
        out_specs=(pl.BlockSpec((1, hw4, C), lambda n: (n, 0, 0)),
                   pl.BlockSpec((1, hw3, C), lambda n: (n, 0, 0)),
                   pl.BlockSpec((1, hw2, C), lambda n: (n, 0, 0)),
                   pl.BlockSpec((1, hw1, C), lambda n: (n, 0, 0)),
                   pl.BlockSpec((1, hw1, C), lambda n: (n, 0, 0))),
        compiler_params=pltpu.CompilerParams(
            dimension_semantics=("parallel",),
            vmem_limit_bytes=56 * 1024 * 1024),
    )(c1, x2, x3, x4, x5,
      r4, r12, r23, r34, wcat, bcat)

    def to_nchw(p, h, w):
        return p.reshape(N, h, w, C).transpose(0, 3, 1, 2)

    return (to_nchw(p1f, H4, W4),
            to_nchw(p2f, H3, W3),
            to_nchw(p3f, H2, W2),
            to_nchw(p4f, H1, W1),
            to_nchw(p5f, H1, W1))


# speedup vs baseline: 1.1406x; 1.0039x over previous
import functools

import numpy as np

import jax
import jax.numpy as jnp
from jax.experimental import pallas as pl
from jax.experimental.pallas import tpu as pltpu

_BF16 = jnp.bfloat16
_F32 = jnp.float32


@functools.lru_cache(maxsize=None)
def _interp_mat_np(out_size, in_size):
    out_size, in_size = int(out_size), int(in_size)
    if out_size == in_size:
        return np.eye(out_size, dtype=np.float32)
    scale = in_size / out_size
    src = np.maximum((np.arange(out_size, dtype=np.float64) + 0.5) * scale - 0.5, 0.0)
    i0 = np.clip(np.floor(src).astype(np.int64), 0, in_size - 1)
    i1 = np.minimum(i0 + 1, in_size - 1)
    frac = (src - i0).astype(np.float32)
    m = np.zeros((out_size, in_size), dtype=np.float32)
    rows = np.arange(out_size)
    np.add.at(m, (rows, i0), 1.0 - frac)
    np.add.at(m, (rows, i1), frac)
    return m


@functools.lru_cache(maxsize=None)
def _resize_lhs_np(in_hw, out_hw):
    (hin, win), (ho, wo) = in_hw, out_hw
    a = _interp_mat_np(int(ho), int(hin))
    b = _interp_mat_np(int(wo), int(win))
    return np.ascontiguousarray(np.kron(a, b))


def _make_body(offs):
    (o0, o1, o2, o3, o4, o5, o6, o7, o8, o9) = offs

    def _neck_body(c1_ref, x2_ref, x3_ref, x4_ref, x5_ref,
                   r4_ref, r12_ref, r23_ref, r34_ref,
                   w_ref, b_ref,
                   p1_ref, p2_ref, p3_ref, p4_ref, p5_ref):
        hw = p5_ref.shape[1]
        y5 = jax.lax.dot_general(
            c1_ref[0].astype(_BF16), w_ref[o0:o1, :],
            (((0,), (0,)), ((), ())), preferred_element_type=_F32)
        y5 = y5.reshape(hw, y5.shape[-1]) + b_ref[0:1, :]
        p5_ref[0] = y5.astype(p5_ref.dtype)

        d = jnp.dot(x2_ref[0].astype(_BF16), w_ref[o1:o2, :],
                    preferred_element_type=_F32) + b_ref[1:2, :]
        y4 = jnp.dot(r4_ref[...], d.astype(_BF16), preferred_element_type=_F32)
        p4_ref[0] = y4.astype(p4_ref.dtype)

        d1 = jnp.dot(x5_ref[0].astype(_BF16), w_ref[o2:o3, :],
                     preferred_element_type=_F32) + b_ref[2:3, :]
        d2 = jnp.dot(x4_ref[0].astype(_BF16), w_ref[o3:o4, :],
                     preferred_element_type=_F32) + b_ref[3:4, :]
        d3 = jnp.dot(x3_ref[0].astype(_BF16), w_ref[o4:o5, :],
                     preferred_element_type=_F32) + b_ref[4:5, :]
        p1 = jnp.dot(r12_ref[...], d1.astype(_BF16), preferred_element_type=_F32)
        d2_2 = jnp.maximum(
            jnp.dot(p1.astype(_BF16), w_ref[o5:o6, :], preferred_element_type=_F32)
            + jnp.dot(d2.astype(_BF16), w_ref[o6:o7, :], preferred_element_type=_F32)
            + b_ref[5:6, :], 0.0)
        p2 = jnp.dot(r23_ref[...], d2_2.astype(_BF16), preferred_element_type=_F32)
        d3_2 = jnp.maximum(
            jnp.dot(p2.astype(_BF16), w_ref[o7:o8, :], preferred_element_type=_F32)
            + jnp.dot(d3.astype(_BF16), w_ref[o8:o9, :], preferred_element_type=_F32)
            + b_ref[6:7, :], 0.0)
        p3 = jnp.dot(r34_ref[...], d3_2.astype(_BF16), preferred_element_type=_F32)
        p1_ref[0] = p1.astype(p1_ref.dtype)
        p2_ref[0] = p2.astype(p2_ref.dtype)
        p3_ref[0] = p3.astype(p3_ref.dtype)

    return _neck_body


def kernel(c1, c2, c3, c4, c5,
           dsn0_w, dsn0_b, dsn1_w, dsn1_b, dsn2_w, dsn2_b, dsn3_w, dsn3_b,
           dsn4_w, dsn4_b, cat0_wa, cat0_wb, cat0_b, cat1_wa, cat1_wb, cat1_b):
    N, ci1, H1, W1 = c1.shape
    _, ci2, H2, W2 = c2.shape
    _, ci3, H3, W3 = c3.shape
    _, ci4, H4, W4 = c4.shape
    _, ci5, H5, W5 = c5.shape
    C = dsn0_w.shape[0]
    dt = c1.dtype
    hw1, hw2, hw3, hw4, hw5 = H1 * W1, H2 * W2, H3 * W3, H4 * W4, H5 * W5

    x2 = c2.transpose(0, 2, 3, 1).reshape(N, hw2, ci2)
    x3 = c3.transpose(0, 2, 3, 1).reshape(N, hw3, ci3)
    x4 = c4.transpose(0, 2, 3, 1).reshape(N, hw4, ci4)
    x5 = c5.transpose(0, 2, 3, 1).reshape(N, hw5, ci5)

    wlist = [dsn0_w, dsn1_w, dsn4_w, dsn3_w, dsn2_w,
             cat0_wa, cat0_wb, cat1_wa, cat1_wb]
    offs = [0]
    for w in wlist:
        offs.append(offs[-1] + w.shape[1])
    wcat = jnp.concatenate([w.T for w in wlist], axis=0).astype(_BF16)
    bcat = jnp.concatenate([dsn0_b.T, dsn1_b.T, dsn4_b.T, dsn3_b.T, dsn2_b.T,
                            cat0_b.T, cat1_b.T], axis=0)

    r4 = jnp.asarray(_resize_lhs_np((H2, W2), (H1, W1))).astype(_BF16)
    r12 = jnp.asarray(_resize_lhs_np((H5, W5), (H4, W4))).astype(_BF16)
    r23 = jnp.asarray(_resize_lhs_np((H4, W4), (H3, W3))).astype(_BF16)
    r34 = jnp.asarray(_resize_lhs_np((H3, W3), (H2, W2))).astype(_BF16)

    def full(shape):
        return pl.BlockSpec(shape, lambda n: (0,) * len(shape))

    p1f, p2f, p3f, p4f, p5f = pl.pallas_call(
        _make_body(tuple(offs)),
        out_shape=(jax.ShapeDtypeStruct((N, hw4, C), dt),
                   jax.ShapeDtypeStruct((N, hw3, C), dt),
                   jax.ShapeDtypeStruct((N, hw2, C), dt),
                   jax.ShapeDtypeStruct((N, hw1, C), dt),
                   jax.ShapeDtypeStruct((N, hw1, C), dt)),
        grid=(N,),
        in_specs=[
            pl.BlockSpec((1, ci1, H1, W1), lambda n: (n, 0, 0, 0)),
            pl.BlockSpec((1, hw2, ci2), lambda n: (n, 0, 0)),
            pl.BlockSpec((1, hw3, ci3), lambda n: (n, 0, 0)),
            pl.BlockSpec((1, hw4, ci4), lambda n: (n, 0, 0)),
            pl.BlockSpec((1, hw5, ci5), lambda n: (n, 0, 0)),
            full((hw1, hw2)), full((hw4, hw5)), full((hw3, hw4)), full((hw2, hw3)),
            full((offs[-1], C)), full((7, C)),
        ],
        out_specs=(pl.BlockSpec((1, hw4, C), lambda n: (n, 0, 0)),
                   pl.BlockSpec((1, hw3, C), lambda n: (n, 0, 0)),
                   pl.BlockSpec((1, hw2, C), lambda n: (n, 0, 0)),
                   pl.BlockSpec((1, hw1, C), lambda n: (n, 0, 0)),
                   pl.BlockSpec((1, hw1, C), lambda n: (n, 0, 0))),
        compiler_params=pltpu.CompilerParams(
            dimension_semantics=("parallel",),
            vmem_limit_bytes=56 * 1024 * 1024),
    )(c1, x2, x3, x4, x5,
      r4, r12, r23, r34, wcat, bcat)

    def to_nchw(p, h, w):
        return p.reshape(N, h, w, C).transpose(0, 3, 1, 2)

    return (to_nchw(p1f, H4, W4),
            to_nchw(p2f, H3, W3),
            to_nchw(p3f, H2, W2),
            to_nchw(p4f, H1, W1),
            to_nchw(p5f, H1, W1))
